# baseline (speedup 1.0000x reference)
import functools

import jax
import jax.numpy as jnp
from jax import lax
from jax.experimental import pallas as pl
from jax.experimental.pallas import tpu as pltpu
from jax.experimental.pallas import tpu_sc as plsc

_N = 50000
_C = 4
_V = 3
_D = 128
_W = _D // 2
_E = _N * _C * _V

_NC = 2
_NS = 16
_NW = _NC * _NS
_R = 128
_EPAD = -(-_E // (_NW * _R)) * (_NW * _R)
_PERW = _EPAD // _NW
_NCHUNK = _PERW // _R

_BN = 2000


def _norm(x):
    n = jnp.sqrt(jnp.sum(x * x, axis=-1, keepdims=True))
    return x / jnp.maximum(n, 1e-12)


def _pack(x):
    xh = x.astype(jnp.bfloat16)
    lo = pltpu.bitcast(xh[:, :_W], jnp.uint16).astype(jnp.uint32)
    hi = pltpu.bitcast(xh[:, _W:], jnp.uint16).astype(jnp.uint32)
    return pltpu.bitcast(lo | (hi << 16), jnp.int32)


def _unpack(w):
    wu = pltpu.bitcast(w, jnp.uint32)
    lo = pltpu.bitcast((wu & 0xFFFF).astype(jnp.uint16), jnp.bfloat16)
    hi = pltpu.bitcast((wu >> 16).astype(jnp.uint16), jnp.bfloat16)
    return jnp.concatenate([lo, hi], axis=1)


def _unpack_pairs(w):
    even = _unpack(w[:, :_W])
    odd = _unpack(w[:, _W:])
    m2 = even.shape[0]
    return jnp.stack([even, odd], axis=1).reshape(2 * m2, _D)


def _prep_body(emb_ref, wneg_ref, bneg_ref, out_ref):
    t0 = _norm(emb_ref[...])
    t1 = (
        jnp.dot(t0.astype(jnp.bfloat16), wneg_ref[...],
                preferred_element_type=jnp.float32)
        + bneg_ref[...]
    )
    out_ref[0] = _pack(t0)
    out_ref[1] = _pack(t1)


def _prep(emb, wneg, bneg2d):
    return pl.pallas_call(
        _prep_body,
        grid=(_N // _BN,),
        in_specs=[
            pl.BlockSpec((_BN, _D), lambda i: (i, 0)),
            pl.BlockSpec((_D, _D), lambda i: (0, 0)),
            pl.BlockSpec((1, _D), lambda i: (0, 0)),
        ],
        out_specs=pl.BlockSpec((2, _BN, _W), lambda i: (0, i, 0)),
        out_shape=jax.ShapeDtypeStruct((2, _N, _W), jnp.int32),
    )(emb, wneg, bneg2d)


_NB = 3


def _sc_gather(table, idx):
    mesh = plsc.VectorSubcoreMesh(core_axis_name="c", subcore_axis_name="s")

    @functools.partial(
        pl.kernel,
        mesh=mesh,
        compiler_params=pltpu.CompilerParams(use_tc_tiling_on_sc=False),
        out_type=jax.ShapeDtypeStruct((_EPAD, _W), jnp.int32),
        scratch_types=[
            pltpu.VMEM((_NB, _R), jnp.int32),
            pltpu.VMEM((_NB, _R, _W), jnp.int32),
        ] + [pltpu.SemaphoreType.DMA] * (2 * _NB),
    )
    def gk(table_hbm, idx_hbm, out_hbm, idx_v, rows_v, *sems):
        sg, ss = sems[:_NB], sems[_NB:]
        wid = lax.axis_index("s") * _NC + lax.axis_index("c")
        base = wid * _PERW

        for b in range(_NB):
            pltpu.sync_copy(idx_hbm.at[pl.ds(base + b * _R, _R)], idx_v.at[b])
            pltpu.async_copy(table_hbm.at[idx_v.at[b]], rows_v.at[b], sg[b])

        def body(g, carry):
            for b in range(_NB):
                off = base + (g * _NB + b) * _R
                pltpu.make_async_copy(
                    table_hbm.at[idx_v.at[b]], rows_v.at[b], sg[b]).wait()
                st = pltpu.async_copy(
                    rows_v.at[b], out_hbm.at[pl.ds(off, _R)], ss[b])
                pltpu.sync_copy(
                    idx_hbm.at[pl.ds(off + _NB * _R, _R)], idx_v.at[b])
                st.wait()
                pltpu.async_copy(table_hbm.at[idx_v.at[b]], rows_v.at[b], sg[b])
            return carry

        lax.fori_loop(0, _NCHUNK // _NB - 1, body, 0)

        for b in range(_NB):
            off = base + (_NCHUNK - _NB + b) * _R
            pltpu.make_async_copy(
                table_hbm.at[idx_v.at[b]], rows_v.at[b], sg[b]).wait()
            pltpu.async_copy(
                rows_v.at[b], out_hbm.at[pl.ds(off, _R)], ss[b]).wait()

    return gk(table, idx)


def _combine_body(final, *refs):
    g_refs = refs[:_C * _V]
    (w1v_ref, b1v_ref, w2v_ref, b2v_ref,
     w1c_ref, b1c_ref, w2c_ref, b2c_ref,
     wneg_ref, bneg_ref, out_ref) = refs[_C * _V:]
    a1 = None
    a2 = None
    for c in range(_C):
        h1 = None
        h2 = None
        for j in range(_V):
            g = _unpack_pairs(g_refs[c * _V + j][...])
            d1 = jnp.dot(g, w1v_ref[j * _D:(j + 1) * _D, :],
                         preferred_element_type=jnp.float32)
            d2 = jnp.dot(g, w2v_ref[j * _D:(j + 1) * _D, :],
                         preferred_element_type=jnp.float32)
            h1 = d1 if h1 is None else h1 + d1
            h2 = d2 if h2 is None else h2 + d2
        ce = _norm(jax.nn.sigmoid(h1 + b1v_ref[...]) + h2 + b2v_ref[...])
        ce = ce.astype(jnp.bfloat16)
        d1 = jnp.dot(ce, w1c_ref[c * _D:(c + 1) * _D, :],
                     preferred_element_type=jnp.float32)
        d2 = jnp.dot(ce, w2c_ref[c * _D:(c + 1) * _D, :],
                     preferred_element_type=jnp.float32)
        a1 = d1 if a1 is None else a1 + d1
        a2 = d2 if a2 is None else a2 + d2
    v = _norm(jax.nn.sigmoid(a1 + b1c_ref[...]) + a2 + b2c_ref[...])
    if final:
        out_ref[...] = v
    else:
        out_ref[0] = _pack(v)
        proj = (
            jnp.dot(v.astype(jnp.bfloat16), wneg_ref[...],
                    preferred_element_type=jnp.float32)
            + bneg_ref[...]
        )
        out_ref[1] = _pack(proj)


def _combine(g, w1v, b1v, w2v, b2v, w1c, b1c, w2c, b2c, wneg, bneg, final):
    def wspec(r):
        return pl.BlockSpec((r, _D), lambda i: (0, 0))

    def gspec(c, j):
        off = (j * _C + c) * (_N // _BN)
        return pl.BlockSpec((_BN // 2, _D), lambda i, o=off: (o + i, 0))

    bspec = pl.BlockSpec((1, _D), lambda i: (0, 0))
    if final:
        out_specs = pl.BlockSpec((_BN, _D), lambda i: (i, 0))
        out_shape = jax.ShapeDtypeStruct((_N, _D), jnp.float32)
    else:
        out_specs = pl.BlockSpec((2, _BN, _W), lambda i: (0, i, 0))
        out_shape = jax.ShapeDtypeStruct((2, _N, _W), jnp.int32)
    return pl.pallas_call(
        functools.partial(_combine_body, final),
        grid=(_N // _BN,),
        in_specs=[gspec(c, j) for c in range(_C) for j in range(_V)] + [
            wspec(_V * _D), bspec, wspec(_V * _D), bspec,
            wspec(_C * _D), bspec, wspec(_C * _D), bspec,
            wspec(_D), bspec,
        ],
        out_specs=out_specs,
        out_shape=out_shape,
    )(*([g] * (_C * _V)),
      w1v, b1v, w2v, b2v, w1c, b1c, w2c, b2c, wneg, bneg)


def kernel(lits, neg, emb, Wneg, bneg, W1v, b1v, W2v, b2v, W1c, b1c, W2c, b2c):
    idx = lits.astype(jnp.int32) + neg.astype(jnp.int32) * _N
    idx = jnp.transpose(idx, (2, 1, 0)).reshape(-1)
    idx = jnp.concatenate([idx, jnp.zeros((_EPAD - _E,), jnp.int32)])

    def r2(v):
        return v.reshape(1, _D)

    bf = jnp.bfloat16
    W1v, W2v, W1c, W2c, Wneg = (w.astype(bf) for w in (W1v, W2v, W1c, W2c, Wneg))

    t = _prep(emb, Wneg, r2(bneg))
    for step in range(2):
        g = _sc_gather(t.reshape(2 * _N, _W), idx)
        g = g.reshape(_EPAD // 2, _D)
        t = _combine(g, W1v, r2(b1v), W2v, r2(b2v),
                     W1c, r2(b1c), W2c, r2(b2c), Wneg, r2(bneg),
                     final=(step == 1))
    return t


# speedup vs baseline: 1.5836x; 1.5836x over previous
import functools

import jax
import jax.numpy as jnp
from jax import lax
from jax.experimental import pallas as pl
from jax.experimental.pallas import tpu as pltpu
from jax.experimental.pallas import tpu_sc as plsc

_N = 50000
_C = 4
_V = 3
_D = 128
_W = _D // 2
_E = _N * _C * _V

_NC = 2
_NS = 16
_NW = _NC * _NS
_R = 128
_EPAD = -(-_E // (_NW * _R)) * (_NW * _R)
_PERW = _EPAD // _NW
_NCHUNK = _PERW // _R

_BN = 2000


def _norm(x):
    n = jnp.sqrt(jnp.sum(x * x, axis=-1, keepdims=True))
    return x / jnp.maximum(n, 1e-12)


def _pack(x):
    xh = x.astype(jnp.bfloat16)
    lo = pltpu.bitcast(xh[:, :_W], jnp.uint16).astype(jnp.uint32)
    hi = pltpu.bitcast(xh[:, _W:], jnp.uint16).astype(jnp.uint32)
    return pltpu.bitcast(lo | (hi << 16), jnp.int32)


def _unpack(w):
    wu = pltpu.bitcast(w, jnp.uint32)
    lo = pltpu.bitcast((wu & 0xFFFF).astype(jnp.uint16), jnp.bfloat16)
    hi = pltpu.bitcast((wu >> 16).astype(jnp.uint16), jnp.bfloat16)
    return jnp.concatenate([lo, hi], axis=1)


def _unpack_pairs(w):
    return jnp.concatenate([_unpack(w[:, :_W]), _unpack(w[:, _W:])], axis=0)


def _prep_body(emb_ref, wneg_ref, bneg_ref, out_ref):
    ev = _norm(emb_ref[:, :_D])
    od = _norm(emb_ref[:, _D:])
    t0 = jnp.concatenate([ev, od], axis=0)
    t1 = (
        jnp.dot(t0.astype(jnp.bfloat16), wneg_ref[...],
                preferred_element_type=jnp.float32)
        + bneg_ref[...]
    )
    out_ref[0] = _pack(t0)
    out_ref[1] = _pack(t1)


def _prep(emb, wneg, bneg2d):
    return pl.pallas_call(
        _prep_body,
        grid=(_N // _BN,),
        in_specs=[
            pl.BlockSpec((_BN // 2, 2 * _D), lambda i: (i, 0)),
            pl.BlockSpec((_D, _D), lambda i: (0, 0)),
            pl.BlockSpec((1, _D), lambda i: (0, 0)),
        ],
        out_specs=pl.BlockSpec((2, _BN, _W), lambda i: (0, i, 0)),
        out_shape=jax.ShapeDtypeStruct((2, _N, _W), jnp.int32),
    )(emb, wneg, bneg2d)


_NB = 3


def _sc_gather(table, idx):
    mesh = plsc.VectorSubcoreMesh(core_axis_name="c", subcore_axis_name="s")

    @functools.partial(
        pl.kernel,
        mesh=mesh,
        compiler_params=pltpu.CompilerParams(use_tc_tiling_on_sc=False),
        out_type=jax.ShapeDtypeStruct((_EPAD, _W), jnp.int32),
        scratch_types=[
            pltpu.VMEM((_NB, _R), jnp.int32),
            pltpu.VMEM((_NB, _R, _W), jnp.int32),
        ] + [pltpu.SemaphoreType.DMA] * (2 * _NB),
    )
    def gk(table_hbm, idx_hbm, out_hbm, idx_v, rows_v, *sems):
        sg, ss = sems[:_NB], sems[_NB:]
        wid = lax.axis_index("s") * _NC + lax.axis_index("c")
        base = wid * _PERW

        for b in range(_NB):
            pltpu.sync_copy(idx_hbm.at[pl.ds(base + b * _R, _R)], idx_v.at[b])
            pltpu.async_copy(table_hbm.at[idx_v.at[b]], rows_v.at[b], sg[b])

        def body(g, carry):
            for b in range(_NB):
                off = base + (g * _NB + b) * _R
                pltpu.make_async_copy(
                    table_hbm.at[idx_v.at[b]], rows_v.at[b], sg[b]).wait()
                st = pltpu.async_copy(
                    rows_v.at[b], out_hbm.at[pl.ds(off, _R)], ss[b])
                pltpu.sync_copy(
                    idx_hbm.at[pl.ds(off + _NB * _R, _R)], idx_v.at[b])
                st.wait()
                pltpu.async_copy(table_hbm.at[idx_v.at[b]], rows_v.at[b], sg[b])
            return carry

        lax.fori_loop(0, _NCHUNK // _NB - 1, body, 0)

        for b in range(_NB):
            off = base + (_NCHUNK - _NB + b) * _R
            pltpu.make_async_copy(
                table_hbm.at[idx_v.at[b]], rows_v.at[b], sg[b]).wait()
            pltpu.async_copy(
                rows_v.at[b], out_hbm.at[pl.ds(off, _R)], ss[b]).wait()

    return gk(table, idx)


def _combine_body(final, *refs):
    g_refs = refs[:_C * _V]
    (w1v_ref, b1v_ref, w2v_ref, b2v_ref,
     w1c_ref, b1c_ref, w2c_ref, b2c_ref,
     wneg_ref, bneg_ref, out_ref) = refs[_C * _V:]
    a1 = None
    a2 = None
    for c in range(_C):
        h1 = None
        h2 = None
        for j in range(_V):
            g = _unpack_pairs(g_refs[c * _V + j][...])
            d1 = jnp.dot(g, w1v_ref[j * _D:(j + 1) * _D, :],
                         preferred_element_type=jnp.float32)
            d2 = jnp.dot(g, w2v_ref[j * _D:(j + 1) * _D, :],
                         preferred_element_type=jnp.float32)
            h1 = d1 if h1 is None else h1 + d1
            h2 = d2 if h2 is None else h2 + d2
        ce = _norm(jax.nn.sigmoid(h1 + b1v_ref[...]) + h2 + b2v_ref[...])
        ce = ce.astype(jnp.bfloat16)
        d1 = jnp.dot(ce, w1c_ref[c * _D:(c + 1) * _D, :],
                     preferred_element_type=jnp.float32)
        d2 = jnp.dot(ce, w2c_ref[c * _D:(c + 1) * _D, :],
                     preferred_element_type=jnp.float32)
        a1 = d1 if a1 is None else a1 + d1
        a2 = d2 if a2 is None else a2 + d2
    v = _norm(jax.nn.sigmoid(a1 + b1c_ref[...]) + a2 + b2c_ref[...])
    if final:
        out_ref[...] = jnp.concatenate(
            [v[:_BN // 2], v[_BN // 2:]], axis=1)
    else:
        out_ref[0] = _pack(v)
        proj = (
            jnp.dot(v.astype(jnp.bfloat16), wneg_ref[...],
                    preferred_element_type=jnp.float32)
            + bneg_ref[...]
        )
        out_ref[1] = _pack(proj)


def _combine(g, w1v, b1v, w2v, b2v, w1c, b1c, w2c, b2c, wneg, bneg, final):
    def wspec(r):
        return pl.BlockSpec((r, _D), lambda i: (0, 0))

    def gspec(c, j):
        off = (j * _C + c) * (_N // _BN)
        return pl.BlockSpec((_BN // 2, _D), lambda i, o=off: (o + i, 0))

    bspec = pl.BlockSpec((1, _D), lambda i: (0, 0))
    if final:
        out_specs = pl.BlockSpec((_BN // 2, 2 * _D), lambda i: (i, 0))
        out_shape = jax.ShapeDtypeStruct((_N // 2, 2 * _D), jnp.float32)
    else:
        out_specs = pl.BlockSpec((2, _BN, _W), lambda i: (0, i, 0))
        out_shape = jax.ShapeDtypeStruct((2, _N, _W), jnp.int32)
    return pl.pallas_call(
        functools.partial(_combine_body, final),
        grid=(_N // _BN,),
        in_specs=[gspec(c, j) for c in range(_C) for j in range(_V)] + [
            wspec(_V * _D), bspec, wspec(_V * _D), bspec,
            wspec(_C * _D), bspec, wspec(_C * _D), bspec,
            wspec(_D), bspec,
        ],
        out_specs=out_specs,
        out_shape=out_shape,
    )(*([g] * (_C * _V)),
      w1v, b1v, w2v, b2v, w1c, b1c, w2c, b2c, wneg, bneg)


def kernel(lits, neg, emb, Wneg, bneg, W1v, b1v, W2v, b2v, W1c, b1c, W2c, b2c):
    l = lits.astype(jnp.int32)
    lperm = (l // _BN) * _BN + (l % _BN) // 2 + (l % 2) * (_BN // 2)
    idx = lperm + neg.astype(jnp.int32) * _N
    idx = jnp.transpose(idx, (2, 1, 0)).reshape(-1)
    idx = jnp.concatenate([idx, jnp.zeros((_EPAD - _E,), jnp.int32)])

    def r2(v):
        return v.reshape(1, _D)

    bf = jnp.bfloat16
    W1v, W2v, W1c, W2c, Wneg = (w.astype(bf) for w in (W1v, W2v, W1c, W2c, Wneg))

    t = _prep(emb.reshape(_N // 2, 2 * _D), Wneg, r2(bneg))
    for step in range(2):
        g = _sc_gather(t.reshape(2 * _N, _W), idx)
        g = g.reshape(_EPAD // 2, _D)
        t = _combine(g, W1v, r2(b1v), W2v, r2(b2v),
                     W1c, r2(b1c), W2c, r2(b2c), Wneg, r2(bneg),
                     final=(step == 1))
    return t.reshape(_N, _D)
